# baseline (speedup 1.0000x reference)
import functools

import jax
import jax.numpy as jnp
from jax import lax
from jax.experimental import pallas as pl
from jax.experimental.pallas import tpu as pltpu
from jax.experimental.pallas import tpu_sc as plsc

L = 16
DIM = 64
WDIM = 128
NUM_WORKERS = 32
CHUNK = 128
_MARGIN = 1.0


def _rsqrt16(x):
    i = lax.bitcast_convert_type(x, jnp.int32)
    i = jnp.int32(0x5F3759DF) - lax.shift_right_arithmetic(i, 1)
    y = lax.bitcast_convert_type(i, jnp.float32)
    for _ in range(3):
        y = y * (1.5 - 0.5 * x * y * y)
    return y


def _tc_relayout(tab_t, num_rows):
    grid = (num_rows + 2 * WDIM - 1) // (2 * WDIM)

    def body(a_ref, b_ref, o_ref):
        o_ref[...] = jnp.concatenate(
            [a_ref[...].T, b_ref[...].T], axis=1)

    return pl.pallas_call(
        body,
        grid=(grid,),
        in_specs=[
            pl.BlockSpec((DIM, WDIM), lambda m: (0, 2 * m)),
            pl.BlockSpec((DIM, WDIM), lambda m: (0, 2 * m + 1)),
        ],
        out_specs=pl.BlockSpec((WDIM, WDIM), lambda m: (m, 0)),
        out_shape=jax.ShapeDtypeStruct((grid * WDIM, WDIM), jnp.float32),
    )(tab_t, tab_t)


def _make_kernel(rows_per_w, nchunk):
    mesh = plsc.VectorSubcoreMesh(core_axis_name="c", subcore_axis_name="s")

    @functools.partial(
        pl.kernel,
        mesh=mesh,
        compiler_params=pltpu.CompilerParams(
            needs_layout_passes=False, use_tc_tiling_on_sc=False),
        out_type=jax.ShapeDtypeStruct((NUM_WORKERS, L), jnp.float32),
        scratch_types=[
            pltpu.VMEM((rows_per_w,), jnp.int32),
            pltpu.VMEM((rows_per_w,), jnp.int32),
            pltpu.VMEM((rows_per_w,), jnp.int32),
            pltpu.VMEM((rows_per_w,), jnp.int32),
            pltpu.VMEM((rows_per_w,), jnp.int32),
            pltpu.VMEM((rows_per_w,), jnp.int32),
            pltpu.VMEM((rows_per_w,), jnp.int32),
            pltpu.VMEM((rows_per_w,), jnp.int32),
            pltpu.VMEM((rows_per_w,), jnp.int32),
            pltpu.VMEM((rows_per_w,), jnp.int32),
            pltpu.VMEM((rows_per_w,), jnp.int32),
            pltpu.VMEM((rows_per_w,), jnp.int32),
            pltpu.VMEM((CHUNK, WDIM), jnp.float32),
            pltpu.VMEM((CHUNK, WDIM), jnp.float32),
            pltpu.VMEM((CHUNK, WDIM), jnp.float32),
            pltpu.VMEM((CHUNK, WDIM), jnp.float32),
            pltpu.VMEM((CHUNK, WDIM), jnp.float32),
            pltpu.VMEM((CHUNK, WDIM), jnp.float32),
            pltpu.VMEM((rows_per_w,), jnp.float32),
            pltpu.VMEM((rows_per_w,), jnp.float32),
            pltpu.VMEM((L,), jnp.float32),
            pltpu.SemaphoreType.DMA,
            pltpu.SemaphoreType.DMA,
        ],
    )
    def transe_sc(ph, pr, pt, nh, nr, nt, ent, rel, out,
                  phidx, pridx, ptidx, nhidx, nridx, ntidx,
                  phcb, prcb, ptcb, nhcb, nrcb, ntcb,
                  hA, rA, tA, hB, rB, tB,
                  psc, nsc, pbuf, semA, semB):
        wid = lax.axis_index("s") * 2 + lax.axis_index("c")
        iota = lax.iota(jnp.int32, L)
        zf = jnp.zeros((L,), jnp.float32)

        for src, dst, cbref in ((ph, phidx, phcb), (pr, pridx, prcb),
                                (pt, ptidx, ptcb), (nh, nhidx, nhcb),
                                (nr, nridx, nrcb), (nt, ntidx, ntcb)):
            pltpu.sync_copy(src.at[pl.ds(wid * rows_per_w, rows_per_w)], dst)
            for g in range(rows_per_w // L):
                s = pl.ds(g * L, L)
                v = dst[s]
                row = lax.shift_left(lax.shift_right_logical(v, 8), 7) | (
                    v & (WDIM - 1))
                cbref[s] = lax.shift_left(
                    lax.shift_right_logical(v, 7) & 1, 6)
                dst[s] = row

        def fire(idxs, seg, bufs, sem):
            hx, rx, tx = idxs
            c = pl.ds(seg * CHUNK, CHUNK)
            return [
                pltpu.async_copy(ent.at[hx.at[c]], bufs[0], sem),
                pltpu.async_copy(rel.at[rx.at[c]], bufs[1], sem),
                pltpu.async_copy(ent.at[tx.at[c]], bufs[2], sem),
            ]

        def compute(bufs, cbs, scref, segbase):
            hrow, rrow, trow = bufs
            hcb, rcb, tcb = cbs

            def blk(b, rowv):
                gv = rowv + segbase
                hc = plsc.load_gather(hcb, [gv])
                rc = plsc.load_gather(rcb, [gv])
                tc = plsc.load_gather(tcb, [gv])

                def p1(u, carry):
                    sh, st, skv = carry
                    for _ in range(16):
                        hv = plsc.load_gather(hrow, [rowv, hc + skv])
                        tv = plsc.load_gather(trow, [rowv, tc + skv])
                        sh = sh + hv * hv
                        st = st + tv * tv
                        skv = (skv + 1) & (DIM - 1)
                    return sh, st, skv

                sh, st, _ = lax.fori_loop(0, DIM // 16, p1, (zf, zf, iota))
                rih = _rsqrt16(sh)
                rit = _rsqrt16(st)

                def p2(u, carry):
                    acc, skv = carry
                    for _ in range(16):
                        hv = plsc.load_gather(hrow, [rowv, hc + skv])
                        rv = plsc.load_gather(rrow, [rowv, rc + skv])
                        tv = plsc.load_gather(trow, [rowv, tc + skv])
                        acc = acc + jnp.abs(hv * rih + rv - tv * rit)
                        skv = (skv + 1) & (DIM - 1)
                    return acc, skv

                acc, _ = lax.fori_loop(0, DIM // 16, p2, (zf, iota))
                plsc.store_scatter(scref, [gv], acc)
                return rowv + L

            lax.fori_loop(0, CHUNK // L, blk, iota)

        A = (hA, rA, tA)
        B = (hB, rB, tB)
        sides = (
            ((phidx, pridx, ptidx), (phcb, prcb, ptcb), psc),
            ((nhidx, nridx, ntidx), (nhcb, nrcb, ntcb), nsc),
        )
        segplan = [(side, seg) for side in range(2) for seg in range(nchunk)]
        pending = [fire(sides[0][0], 0, A, semA),
                   fire(sides[0][0], 1, B, semB)]
        for i, (side, seg) in enumerate(segplan):
            bufs, sem = (A, semA) if i % 2 == 0 else (B, semB)
            for c in pending.pop(0):
                c.wait()
            idxs, cbs, scref = sides[side]
            compute(bufs, cbs, scref, seg * CHUNK)
            j = i + 2
            if j < len(segplan):
                nside, nseg = segplan[j]
                pending.append(fire(sides[nside][0], nseg, bufs, sem))

        accv = zf
        for b in range(rows_per_w // L):
            p = psc[pl.ds(b * L, L)]
            n = nsc[pl.ds(b * L, L)]
            accv = accv + jnp.maximum(p - n + _MARGIN, 0.0)
        pbuf[...] = accv
        pltpu.sync_copy(pbuf, out.at[wid])

    return transe_sc


def kernel(batch_positives, batch_negatives, entity_emb, relation_emb):
    batch = batch_positives.shape[0]
    rows_per_w = batch // NUM_WORKERS
    nchunk = rows_per_w // CHUNK

    ent_lin = _tc_relayout(entity_emb.T, entity_emb.shape[0])
    rel_lin = _tc_relayout(relation_emb.T, relation_emb.shape[0])

    partials = _make_kernel(rows_per_w, nchunk)(
        batch_positives[:, 0], batch_positives[:, 1], batch_positives[:, 2],
        batch_negatives[:, 0], batch_negatives[:, 1], batch_negatives[:, 2],
        ent_lin, rel_lin)
    return jnp.sum(partials) / jnp.float32(batch)


# speedup vs baseline: 2.2480x; 2.2480x over previous
import functools

import jax
import jax.numpy as jnp
from jax import lax
from jax.experimental import pallas as pl
from jax.experimental.pallas import tpu as pltpu
from jax.experimental.pallas import tpu_sc as plsc

L = 16
DIM = 64
WDIM = 128
NUM_WORKERS = 32
CHUNK = 128
_MARGIN = 1.0


def _rsqrt16(x):
    i = lax.bitcast_convert_type(x, jnp.int32)
    i = jnp.int32(0x5F3759DF) - lax.shift_right_arithmetic(i, 1)
    y = lax.bitcast_convert_type(i, jnp.float32)
    for _ in range(3):
        y = y * (1.5 - 0.5 * x * y * y)
    return y


SEG = 512


def _tc_relayout(tab_t, num_rows):
    grid = (num_rows + 2 * SEG - 1) // (2 * SEG)

    def body(a_ref, b_ref, o_ref):
        i2d = jnp.equal(
            lax.broadcasted_iota(jnp.int32, (DIM, DIM), 0),
            lax.broadcasted_iota(jnp.int32, (DIM, DIM), 1),
        ).astype(jnp.float32)
        dn = (((0,), (0,)), ((), ()))
        ya = lax.dot_general(a_ref[...], i2d, dn,
                             preferred_element_type=jnp.float32)
        yb = lax.dot_general(b_ref[...], i2d, dn,
                             preferred_element_type=jnp.float32)
        o_ref[...] = jnp.concatenate([ya, yb], axis=1)

    return pl.pallas_call(
        body,
        grid=(grid,),
        in_specs=[
            pl.BlockSpec((DIM, SEG), lambda m: (0, 2 * m)),
            pl.BlockSpec((DIM, SEG), lambda m: (0, 2 * m + 1)),
        ],
        out_specs=pl.BlockSpec((SEG, WDIM), lambda m: (m, 0)),
        out_shape=jax.ShapeDtypeStruct((grid * SEG, WDIM), jnp.float32),
    )(tab_t, tab_t)


def _make_kernel(rows_per_w, nchunk):
    mesh = plsc.VectorSubcoreMesh(core_axis_name="c", subcore_axis_name="s")

    @functools.partial(
        pl.kernel,
        mesh=mesh,
        compiler_params=pltpu.CompilerParams(
            needs_layout_passes=False, use_tc_tiling_on_sc=False),
        out_type=jax.ShapeDtypeStruct((NUM_WORKERS, L), jnp.float32),
        scratch_types=[
            pltpu.VMEM((rows_per_w,), jnp.int32),
            pltpu.VMEM((rows_per_w,), jnp.int32),
            pltpu.VMEM((rows_per_w,), jnp.int32),
            pltpu.VMEM((rows_per_w,), jnp.int32),
            pltpu.VMEM((rows_per_w,), jnp.int32),
            pltpu.VMEM((rows_per_w,), jnp.int32),
            pltpu.VMEM((rows_per_w,), jnp.int32),
            pltpu.VMEM((rows_per_w,), jnp.int32),
            pltpu.VMEM((rows_per_w,), jnp.int32),
            pltpu.VMEM((rows_per_w,), jnp.int32),
            pltpu.VMEM((rows_per_w,), jnp.int32),
            pltpu.VMEM((rows_per_w,), jnp.int32),
            pltpu.VMEM((CHUNK, WDIM), jnp.float32),
            pltpu.VMEM((CHUNK, WDIM), jnp.float32),
            pltpu.VMEM((CHUNK, WDIM), jnp.float32),
            pltpu.VMEM((CHUNK, WDIM), jnp.float32),
            pltpu.VMEM((CHUNK, WDIM), jnp.float32),
            pltpu.VMEM((CHUNK, WDIM), jnp.float32),
            pltpu.VMEM((rows_per_w,), jnp.float32),
            pltpu.VMEM((rows_per_w,), jnp.float32),
            pltpu.VMEM((L,), jnp.float32),
            pltpu.SemaphoreType.DMA,
            pltpu.SemaphoreType.DMA,
        ],
    )
    def transe_sc(ph, pr, pt, nh, nr, nt, ent, rel, out,
                  phidx, pridx, ptidx, nhidx, nridx, ntidx,
                  phcb, prcb, ptcb, nhcb, nrcb, ntcb,
                  hA, rA, tA, hB, rB, tB,
                  psc, nsc, pbuf, semA, semB):
        wid = lax.axis_index("s") * 2 + lax.axis_index("c")
        iota = lax.iota(jnp.int32, L)
        zf = jnp.zeros((L,), jnp.float32)

        for src, dst, cbref in ((ph, phidx, phcb), (pr, pridx, prcb),
                                (pt, ptidx, ptcb), (nh, nhidx, nhcb),
                                (nr, nridx, nrcb), (nt, ntidx, ntcb)):
            pltpu.sync_copy(src.at[pl.ds(wid * rows_per_w, rows_per_w)], dst)
            for g in range(rows_per_w // L):
                s = pl.ds(g * L, L)
                v = dst[s]
                row = lax.shift_left(lax.shift_right_logical(v, 10), 9) | (
                    v & (SEG - 1))
                cbref[s] = lax.shift_left(
                    lax.shift_right_logical(v, 9) & 1, 6)
                dst[s] = row

        def fire(idxs, seg, bufs, sem):
            hx, rx, tx = idxs
            c = pl.ds(seg * CHUNK, CHUNK)
            return [
                pltpu.async_copy(ent.at[hx.at[c]], bufs[0], sem),
                pltpu.async_copy(rel.at[rx.at[c]], bufs[1], sem),
                pltpu.async_copy(ent.at[tx.at[c]], bufs[2], sem),
            ]

        def compute(bufs, cbs, scref, segbase):
            hrow, rrow, trow = bufs
            hcb, rcb, tcb = cbs

            def blk(b, rowv):
                gv = rowv + segbase
                hc = plsc.load_gather(hcb, [gv])
                rc = plsc.load_gather(rcb, [gv])
                tc = plsc.load_gather(tcb, [gv])

                def p1(u, carry):
                    sh, st, skv = carry
                    for _ in range(16):
                        hv = plsc.load_gather(hrow, [rowv, hc + skv])
                        tv = plsc.load_gather(trow, [rowv, tc + skv])
                        sh = sh + hv * hv
                        st = st + tv * tv
                        skv = (skv + 1) & (DIM - 1)
                    return sh, st, skv

                sh, st, _ = lax.fori_loop(0, DIM // 16, p1, (zf, zf, iota))
                rih = _rsqrt16(sh)
                rit = _rsqrt16(st)

                def p2(u, carry):
                    acc, skv = carry
                    for _ in range(16):
                        hv = plsc.load_gather(hrow, [rowv, hc + skv])
                        rv = plsc.load_gather(rrow, [rowv, rc + skv])
                        tv = plsc.load_gather(trow, [rowv, tc + skv])
                        acc = acc + jnp.abs(hv * rih + rv - tv * rit)
                        skv = (skv + 1) & (DIM - 1)
                    return acc, skv

                acc, _ = lax.fori_loop(0, DIM // 16, p2, (zf, iota))
                plsc.store_scatter(scref, [gv], acc)
                return rowv + L

            lax.fori_loop(0, CHUNK // L, blk, iota)

        A = (hA, rA, tA)
        B = (hB, rB, tB)
        sides = (
            ((phidx, pridx, ptidx), (phcb, prcb, ptcb), psc),
            ((nhidx, nridx, ntidx), (nhcb, nrcb, ntcb), nsc),
        )
        segplan = [(side, seg) for side in range(2) for seg in range(nchunk)]
        pending = [fire(sides[0][0], 0, A, semA),
                   fire(sides[0][0], 1, B, semB)]
        for i, (side, seg) in enumerate(segplan):
            bufs, sem = (A, semA) if i % 2 == 0 else (B, semB)
            for c in pending.pop(0):
                c.wait()
            idxs, cbs, scref = sides[side]
            compute(bufs, cbs, scref, seg * CHUNK)
            j = i + 2
            if j < len(segplan):
                nside, nseg = segplan[j]
                pending.append(fire(sides[nside][0], nseg, bufs, sem))

        accv = zf
        for b in range(rows_per_w // L):
            p = psc[pl.ds(b * L, L)]
            n = nsc[pl.ds(b * L, L)]
            accv = accv + jnp.maximum(p - n + _MARGIN, 0.0)
        pbuf[...] = accv
        pltpu.sync_copy(pbuf, out.at[wid])

    return transe_sc


def kernel(batch_positives, batch_negatives, entity_emb, relation_emb):
    batch = batch_positives.shape[0]
    rows_per_w = batch // NUM_WORKERS
    nchunk = rows_per_w // CHUNK

    ent_lin = _tc_relayout(entity_emb.T, entity_emb.shape[0])
    rel_lin = _tc_relayout(relation_emb.T, relation_emb.shape[0])

    partials = _make_kernel(rows_per_w, nchunk)(
        batch_positives[:, 0], batch_positives[:, 1], batch_positives[:, 2],
        batch_negatives[:, 0], batch_negatives[:, 1], batch_negatives[:, 2],
        ent_lin, rel_lin)
    return jnp.sum(partials) / jnp.float32(batch)


# speedup vs baseline: 2.2774x; 1.0131x over previous
import functools

import jax
import jax.numpy as jnp
from jax import lax
from jax.experimental import pallas as pl
from jax.experimental.pallas import tpu as pltpu
from jax.experimental.pallas import tpu_sc as plsc

L = 16
DIM = 64
NUM_WORKERS = 32
CHUNK = 128
HALF = 256
_MARGIN = 1.0


def _rsqrt16(x):
    i = lax.bitcast_convert_type(x, jnp.int32)
    i = jnp.int32(0x5F3759DF) - lax.shift_right_arithmetic(i, 1)
    y = lax.bitcast_convert_type(i, jnp.float32)
    for _ in range(3):
        y = y * (1.5 - 0.5 * x * y * y)
    return y


TSEG = 1024


def _tc_relayout(tab_t, pe, po, num_rows):
    grid = (num_rows + TSEG - 1) // TSEG

    def body(a_ref, pe_ref, po_ref, o_ref):
        dn = (((1,), (1,)), ((), ()))
        ya = lax.dot_general(pe_ref[...], a_ref[...], dn,
                             preferred_element_type=jnp.float32)
        yb = lax.dot_general(po_ref[...], a_ref[...], dn,
                             preferred_element_type=jnp.float32)
        o_ref[...] = jnp.concatenate([ya, yb], axis=1)

    return pl.pallas_call(
        body,
        grid=(grid,),
        in_specs=[
            pl.BlockSpec((DIM, TSEG), lambda m: (0, m)),
            pl.BlockSpec((TSEG // 2, TSEG), lambda m: (0, 0)),
            pl.BlockSpec((TSEG // 2, TSEG), lambda m: (0, 0)),
        ],
        out_specs=pl.BlockSpec((TSEG // 2, WDIM), lambda m: (m, 0)),
        out_shape=jax.ShapeDtypeStruct((grid * TSEG // 2, WDIM), jnp.float32),
    )(tab_t, pe, po)


WDIM = 128


def _make_kernel(rows_per_w, nchunk):
    mesh = plsc.VectorSubcoreMesh(core_axis_name="c", subcore_axis_name="s")

    @functools.partial(
        pl.kernel,
        mesh=mesh,
        compiler_params=pltpu.CompilerParams(
            needs_layout_passes=False, use_tc_tiling_on_sc=False),
        out_type=jax.ShapeDtypeStruct((NUM_WORKERS, L), jnp.float32),
        scratch_types=[
            pltpu.VMEM((rows_per_w,), jnp.int32),
            pltpu.VMEM((rows_per_w,), jnp.int32),
            pltpu.VMEM((rows_per_w,), jnp.int32),
            pltpu.VMEM((rows_per_w,), jnp.int32),
            pltpu.VMEM((rows_per_w,), jnp.int32),
            pltpu.VMEM((rows_per_w,), jnp.int32),
            pltpu.VMEM((HALF, DIM), jnp.float32),
            pltpu.VMEM((HALF, DIM), jnp.float32),
            pltpu.VMEM((HALF, DIM), jnp.float32),
            pltpu.VMEM((HALF, DIM), jnp.float32),
            pltpu.VMEM((HALF, DIM), jnp.float32),
            pltpu.VMEM((HALF, DIM), jnp.float32),
            pltpu.VMEM((rows_per_w,), jnp.float32),
            pltpu.VMEM((rows_per_w,), jnp.float32),
            pltpu.VMEM((L,), jnp.float32),
            pltpu.SemaphoreType.DMA,
            pltpu.SemaphoreType.DMA,
        ],
    )
    def transe_sc(ph, pr, pt, nh, nr, nt, ent, rel, out,
                  phidx, pridx, ptidx, nhidx, nridx, ntidx,
                  hA, rA, tA, hB, rB, tB,
                  psc, nsc, pbuf, semA, semB):
        wid = lax.axis_index("s") * 2 + lax.axis_index("c")
        iota = lax.iota(jnp.int32, L)
        zf = jnp.zeros((L,), jnp.float32)
        zi = jnp.zeros((L,), jnp.int32)

        for src, dst in ((ph, phidx), (pr, pridx), (pt, ptidx),
                         (nh, nhidx), (nr, nridx), (nt, ntidx)):
            pltpu.sync_copy(src.at[pl.ds(wid * rows_per_w, rows_per_w)], dst)

        def fire(hx, rx, tx, half, bufs, sem):
            cps = []
            for k in range(HALF // CHUNK):
                c = pl.ds((half * (HALF // CHUNK) + k) * CHUNK, CHUNK)
                d = pl.ds(k * CHUNK, CHUNK)
                cps.append(pltpu.async_copy(ent.at[hx.at[c]], bufs[0].at[d], sem))
                cps.append(pltpu.async_copy(rel.at[rx.at[c]], bufs[1].at[d], sem))
                cps.append(pltpu.async_copy(ent.at[tx.at[c]], bufs[2].at[d], sem))
            return cps

        def compute(bufs, scref, base):
            hrow, rrow, trow = bufs

            def blk(b, rowv):
                def p1(u, carry):
                    sh, st, colv = carry
                    for _ in range(16):
                        hv = plsc.load_gather(hrow, [rowv, colv])
                        tv = plsc.load_gather(trow, [rowv, colv])
                        sh = sh + hv * hv
                        st = st + tv * tv
                        colv = (colv + 1) & (DIM - 1)
                    return sh, st, colv

                sh, st, _ = lax.fori_loop(0, DIM // 16, p1, (zf, zf, iota))
                rih = _rsqrt16(sh)
                rit = _rsqrt16(st)

                def p2(u, carry):
                    acc, colv = carry
                    for _ in range(16):
                        hv = plsc.load_gather(hrow, [rowv, colv])
                        rv = plsc.load_gather(rrow, [rowv, colv])
                        tv = plsc.load_gather(trow, [rowv, colv])
                        acc = acc + jnp.abs(hv * rih + rv - tv * rit)
                        colv = (colv + 1) & (DIM - 1)
                    return acc, colv

                acc, _ = lax.fori_loop(0, DIM // 16, p2, (zf, iota))
                plsc.store_scatter(scref, [rowv + base], acc)
                return rowv + L

            lax.fori_loop(0, HALF // L, blk, iota)

        A = (hA, rA, tA)
        B = (hB, rB, tB)
        pending = [fire(phidx, pridx, ptidx, 0, A, semA),
                   fire(phidx, pridx, ptidx, 1, B, semB)]
        plan = [
            (A, psc, 0, (nhidx, nridx, ntidx, 0, A, semA)),
            (B, psc, HALF, (nhidx, nridx, ntidx, 1, B, semB)),
            (A, nsc, 0, None),
            (B, nsc, HALF, None),
        ]
        for bufs, scref, base, refire in plan:
            for c in pending.pop(0):
                c.wait()
            compute(bufs, scref, base)
            if refire is not None:
                pending.append(fire(*refire))

        accv = zf
        for b in range(rows_per_w // L):
            p = psc[pl.ds(b * L, L)]
            n = nsc[pl.ds(b * L, L)]
            accv = accv + jnp.maximum(p - n + _MARGIN, 0.0)
        pbuf[...] = accv
        pltpu.sync_copy(pbuf, out.at[wid])

    return transe_sc


def kernel(batch_positives, batch_negatives, entity_emb, relation_emb):
    batch = batch_positives.shape[0]
    rows_per_w = batch // NUM_WORKERS
    nchunk = rows_per_w // CHUNK

    def split(b):
        return b[:, 0], b[:, 1], b[:, 2]

    ph, pr, pt = split(batch_positives)
    nh, nr, nt = split(batch_negatives)
    k = jnp.arange(TSEG // 2, dtype=jnp.int32)[:, None]
    e = jnp.arange(TSEG, dtype=jnp.int32)[None, :]
    pe = (e == 2 * k).astype(jnp.float32)
    po = (e == 2 * k + 1).astype(jnp.float32)
    ent_lin = _tc_relayout(entity_emb.T, pe, po,
                           entity_emb.shape[0]).reshape(-1, DIM)
    rel_lin = _tc_relayout(relation_emb.T, pe, po,
                           relation_emb.shape[0]).reshape(-1, DIM)
    partials = _make_kernel(rows_per_w, nchunk)(
        ph, pr, pt, nh, nr, nt, ent_lin, rel_lin)
    return jnp.sum(partials) / jnp.float32(batch)


# speedup vs baseline: 3.3968x; 1.4915x over previous
import functools

import jax
import jax.numpy as jnp
from jax import lax
from jax.experimental import pallas as pl
from jax.experimental.pallas import tpu as pltpu
from jax.experimental.pallas import tpu_sc as plsc

L = 16
DIM = 64
NUM_WORKERS = 32
CHUNK = 128
HALF = 256
_MARGIN = 1.0


def _rsqrt16(x):
    i = lax.bitcast_convert_type(x, jnp.int32)
    i = jnp.int32(0x5F3759DF) - lax.shift_right_arithmetic(i, 1)
    y = lax.bitcast_convert_type(i, jnp.float32)
    for _ in range(3):
        y = y * (1.5 - 0.5 * x * y * y)
    return y


def _make_kernel(rows_per_w, nchunk):
    mesh = plsc.VectorSubcoreMesh(core_axis_name="c", subcore_axis_name="s")

    @functools.partial(
        pl.kernel,
        mesh=mesh,
        compiler_params=pltpu.CompilerParams(
            needs_layout_passes=False, use_tc_tiling_on_sc=False),
        out_type=jax.ShapeDtypeStruct((NUM_WORKERS, L), jnp.float32),
        scratch_types=[
            pltpu.VMEM((nchunk, CHUNK), jnp.int32),
            pltpu.VMEM((nchunk, CHUNK), jnp.int32),
            pltpu.VMEM((nchunk, CHUNK), jnp.int32),
            pltpu.VMEM((nchunk, CHUNK), jnp.int32),
            pltpu.VMEM((nchunk, CHUNK), jnp.int32),
            pltpu.VMEM((nchunk, CHUNK), jnp.int32),
            pltpu.VMEM((HALF, DIM), jnp.float32),
            pltpu.VMEM((HALF, DIM), jnp.float32),
            pltpu.VMEM((HALF, DIM), jnp.float32),
            pltpu.VMEM((HALF, DIM), jnp.float32),
            pltpu.VMEM((HALF, DIM), jnp.float32),
            pltpu.VMEM((HALF, DIM), jnp.float32),
            pltpu.VMEM((rows_per_w,), jnp.float32),
            pltpu.VMEM((rows_per_w,), jnp.float32),
            pltpu.VMEM((L,), jnp.float32),
            pltpu.SemaphoreType.DMA,
            pltpu.SemaphoreType.DMA,
        ],
    )
    def transe_sc(ph, pr, pt, nh, nr, nt, ent, rel, out,
                  phidx, pridx, ptidx, nhidx, nridx, ntidx,
                  hA, rA, tA, hB, rB, tB,
                  psc, nsc, pbuf, semA, semB):
        wid = lax.axis_index("s") * 2 + lax.axis_index("c")
        iota = lax.iota(jnp.int32, L)
        zf = jnp.zeros((L,), jnp.float32)
        zi = jnp.zeros((L,), jnp.int32)

        for src, dst in ((ph, phidx), (pr, pridx), (pt, ptidx),
                         (nh, nhidx), (nr, nridx), (nt, ntidx)):
            pltpu.sync_copy(src.at[wid], dst)

        def fire(hx, rx, tx, half, bufs, sem):
            cps = []
            for k in range(HALF // CHUNK):
                c = half * (HALF // CHUNK) + k
                d = pl.ds(k * CHUNK, CHUNK)
                cps.append(pltpu.async_copy(ent.at[hx.at[c]], bufs[0].at[d], sem))
                cps.append(pltpu.async_copy(rel.at[rx.at[c]], bufs[1].at[d], sem))
                cps.append(pltpu.async_copy(ent.at[tx.at[c]], bufs[2].at[d], sem))
            return cps

        def compute(bufs, scref, base):
            hrow, rrow, trow = bufs

            def blk(b, rowv):
                def p1(u, carry):
                    sh, st, colv = carry
                    for _ in range(16):
                        hv = plsc.load_gather(hrow, [rowv, colv])
                        tv = plsc.load_gather(trow, [rowv, colv])
                        sh = sh + hv * hv
                        st = st + tv * tv
                        colv = (colv + 1) & (DIM - 1)
                    return sh, st, colv

                sh, st, _ = lax.fori_loop(0, DIM // 16, p1, (zf, zf, iota))
                rih = _rsqrt16(sh)
                rit = _rsqrt16(st)

                def p2(u, carry):
                    acc, colv = carry
                    for _ in range(16):
                        hv = plsc.load_gather(hrow, [rowv, colv])
                        rv = plsc.load_gather(rrow, [rowv, colv])
                        tv = plsc.load_gather(trow, [rowv, colv])
                        acc = acc + jnp.abs(hv * rih + rv - tv * rit)
                        colv = (colv + 1) & (DIM - 1)
                    return acc, colv

                acc, _ = lax.fori_loop(0, DIM // 16, p2, (zf, iota))
                plsc.store_scatter(scref, [rowv + base], acc)
                return rowv + L

            lax.fori_loop(0, HALF // L, blk, iota)

        A = (hA, rA, tA)
        B = (hB, rB, tB)
        pending = [fire(phidx, pridx, ptidx, 0, A, semA),
                   fire(phidx, pridx, ptidx, 1, B, semB)]
        plan = [
            (A, psc, 0, (nhidx, nridx, ntidx, 0, A, semA)),
            (B, psc, HALF, (nhidx, nridx, ntidx, 1, B, semB)),
            (A, nsc, 0, None),
            (B, nsc, HALF, None),
        ]
        for bufs, scref, base, refire in plan:
            for c in pending.pop(0):
                c.wait()
            compute(bufs, scref, base)
            if refire is not None:
                pending.append(fire(*refire))

        accv = zf
        for b in range(rows_per_w // L):
            p = psc[pl.ds(b * L, L)]
            n = nsc[pl.ds(b * L, L)]
            accv = accv + jnp.maximum(p - n + _MARGIN, 0.0)
        pbuf[...] = accv
        pltpu.sync_copy(pbuf, out.at[wid])

    return transe_sc


def kernel(batch_positives, batch_negatives, entity_emb, relation_emb):
    batch = batch_positives.shape[0]
    rows_per_w = batch // NUM_WORKERS
    nchunk = rows_per_w // CHUNK

    def split(b):
        return (b[:, 0].reshape(NUM_WORKERS, nchunk, CHUNK),
                b[:, 1].reshape(NUM_WORKERS, nchunk, CHUNK),
                b[:, 2].reshape(NUM_WORKERS, nchunk, CHUNK))

    ph, pr, pt = split(batch_positives)
    nh, nr, nt = split(batch_negatives)
    partials = _make_kernel(rows_per_w, nchunk)(
        ph, pr, pt, nh, nr, nt, entity_emb, relation_emb)
    return jnp.sum(partials) / jnp.float32(batch)
